# baseline (speedup 1.0000x reference)
import functools

import jax
import jax.numpy as jnp
from jax import lax
from jax.experimental import pallas as pl
from jax.experimental.pallas import tpu as pltpu
from jax.experimental.pallas import tpu_sc as plsc

K = 100000
D = 16
B = 16384
RG = 8
G = K // RG

NC = 2
NS = 16
NW = NC * NS
CH = B // NW
GCH = 128
NG = CH // GCH


def _sc_logreg(xt_hbm, a_hbm, m_hbm, out_hbm, idx_v, gidx_v, grp_v, xt_v,
               out_v, sem):
    cid = lax.axis_index("c")
    sid = lax.axis_index("s")
    wid = sid * NC + cid
    base = wid * CH

    pltpu.sync_copy(a_hbm.at[pl.ds(base, CH)], idx_v)
    for i in range(CH // 16):
        gidx_v[pl.ds(i * 16, 16)] = lax.shift_right_logical(
            idx_v[pl.ds(i * 16, 16)], 3
        )

    copies = []
    for j in range(NG):
        copies.append(
            pltpu.async_copy(
                m_hbm.at[gidx_v.at[pl.ds(j * GCH, GCH)]],
                grp_v.at[pl.ds(j * GCH, GCH)],
                sem,
            )
        )
    pltpu.sync_copy(xt_hbm.at[:, pl.ds(base, CH)], xt_v)
    for cp in copies:
        cp.wait()

    lanes = lax.iota(jnp.int32, 16)

    def block(blk, _):
        row_ids = blk * 16 + lanes
        idx16 = idx_v[pl.ds(blk * 16, 16)]
        off = (idx16 & 7) * D
        acc = jnp.zeros((16,), jnp.float32)
        for d in range(D):
            xv = xt_v[d, pl.ds(blk * 16, 16)]
            gv = plsc.load_gather(grp_v, [row_ids, off + d])
            acc = acc + xv * gv
        out_v[pl.ds(blk * 16, 16)] = 1.0 / (1.0 + jnp.exp(-acc))
        return _

    lax.fori_loop(0, CH // 16, block, 0)

    pltpu.sync_copy(out_v, out_hbm.at[pl.ds(base, CH)])


@functools.partial(
    pl.kernel,
    mesh=plsc.VectorSubcoreMesh(core_axis_name="c", subcore_axis_name="s"),
    compiler_params=pltpu.CompilerParams(
        needs_layout_passes=False, use_tc_tiling_on_sc=True
    ),
    out_type=jax.ShapeDtypeStruct((B,), jnp.float32),
    scratch_types=[
        pltpu.VMEM((CH,), jnp.int32),
        pltpu.VMEM((CH,), jnp.int32),
        pltpu.VMEM((CH, RG * D), jnp.float32),
        pltpu.VMEM((D, CH), jnp.float32),
        pltpu.VMEM((CH,), jnp.float32),
        pltpu.SemaphoreType.DMA,
    ],
)
def _logreg_kernel(xt_hbm, a_hbm, m_hbm, out_hbm, idx_v, gidx_v, grp_v, xt_v,
                   out_v, sem):
    _sc_logreg(xt_hbm, a_hbm, m_hbm, out_hbm, idx_v, gidx_v, grp_v, xt_v,
               out_v, sem)


def kernel(X, A, m):
    return _logreg_kernel(
        X.T, A.astype(jnp.int32), m.reshape(G, RG * D)
    )


# speedup vs baseline: 1.0146x; 1.0146x over previous
import functools

import jax
import jax.numpy as jnp
from jax import lax
from jax.experimental import pallas as pl
from jax.experimental.pallas import tpu as pltpu
from jax.experimental.pallas import tpu_sc as plsc

K = 100000
D = 16
B = 16384
RG = 8
G = K // RG

NC = 2
NS = 16
NW = NC * NS
CH = B // NW
GCH = 128
NG = CH // GCH


def _sc_logreg(xt_hbm, a_hbm, m_hbm, out_hbm, idx_v, gidx_v, grp_v, xt_v,
               out_v, sem):
    cid = lax.axis_index("c")
    sid = lax.axis_index("s")
    wid = sid * NC + cid
    base = wid * CH

    pltpu.sync_copy(a_hbm.at[pl.ds(base, CH)], idx_v)
    for i in range(CH // 16):
        gidx_v[pl.ds(i * 16, 16)] = lax.shift_right_logical(
            idx_v[pl.ds(i * 16, 16)], 3
        )

    copies = []
    for j in range(NG):
        copies.append(
            pltpu.async_copy(
                m_hbm.at[gidx_v.at[pl.ds(j * GCH, GCH)]],
                grp_v.at[pl.ds(j * GCH, GCH)],
                sem,
            )
        )
    pltpu.sync_copy(xt_hbm.at[:, pl.ds(base, CH)], xt_v)

    lanes = lax.iota(jnp.int32, 16)

    def block(blk, _):
        row_ids = blk * 16 + lanes
        idx16 = idx_v[pl.ds(blk * 16, 16)]
        off = (idx16 & 7) * D
        acc = jnp.zeros((16,), jnp.float32)
        for d in range(D):
            xv = xt_v[d, pl.ds(blk * 16, 16)]
            gv = plsc.load_gather(grp_v, [row_ids, off + d])
            acc = acc + xv * gv
        out_v[pl.ds(blk * 16, 16)] = 1.0 / (1.0 + jnp.exp(-acc))
        return _

    for j in range(NG):
        copies[j].wait()
        lax.fori_loop(j * (GCH // 16), (j + 1) * (GCH // 16), block, 0)

    pltpu.sync_copy(out_v, out_hbm.at[pl.ds(base, CH)])


@functools.partial(
    pl.kernel,
    mesh=plsc.VectorSubcoreMesh(core_axis_name="c", subcore_axis_name="s"),
    compiler_params=pltpu.CompilerParams(
        needs_layout_passes=False, use_tc_tiling_on_sc=True
    ),
    out_type=jax.ShapeDtypeStruct((B,), jnp.float32),
    scratch_types=[
        pltpu.VMEM((CH,), jnp.int32),
        pltpu.VMEM((CH,), jnp.int32),
        pltpu.VMEM((CH, RG * D), jnp.float32),
        pltpu.VMEM((D, CH), jnp.float32),
        pltpu.VMEM((CH,), jnp.float32),
        pltpu.SemaphoreType.DMA,
    ],
)
def _logreg_kernel(xt_hbm, a_hbm, m_hbm, out_hbm, idx_v, gidx_v, grp_v, xt_v,
                   out_v, sem):
    _sc_logreg(xt_hbm, a_hbm, m_hbm, out_hbm, idx_v, gidx_v, grp_v, xt_v,
               out_v, sem)


def kernel(X, A, m):
    return _logreg_kernel(
        X.T, A.astype(jnp.int32), m.reshape(G, RG * D)
    )
